# baseline (speedup 1.0000x reference)
import functools

import jax
import jax.numpy as jnp
from jax import lax
from jax.experimental import pallas as pl
from jax.experimental.pallas import tpu as pltpu
from jax.experimental.pallas import tpu_sc as plsc

NC = 2
NS = 16
LN = 16
NW = NC * NS

CH2 = 96
CH4 = 128
RING = 3


def _tc_encode(x, W_enc, b_enc):
    def body(x_ref, w_ref, b_ref, o_ref):
        o_ref[...] = (
            jnp.dot(x_ref[...], w_ref[...], preferred_element_type=jnp.float32)
            + b_ref[...]
        )

    return pl.pallas_call(
        body,
        out_shape=jax.ShapeDtypeStruct(x.shape, jnp.float32),
    )(x, W_enc, b_enc.reshape(1, -1))


def _tc_node_mlp(p, h, beta, W1a, W1b, b1):
    n, d = h.shape
    hid = W1a.shape[1]

    def body(p_ref, h_ref, beta_ref, wa_ref, wb_ref, b1_ref, ab_ref):
        ne = jnp.maximum(p_ref[0] + p_ref[1] + beta_ref[0] * h_ref[...], 0.0)
        a = jnp.dot(ne, wa_ref[...], preferred_element_type=jnp.float32) + b1_ref[...]
        b = jnp.dot(ne, wb_ref[...], preferred_element_type=jnp.float32)
        ab_ref[...] = jnp.concatenate([a, b], axis=1)

    return pl.pallas_call(
        body,
        in_specs=[
            pl.BlockSpec(memory_space=pltpu.VMEM),
            pl.BlockSpec(memory_space=pltpu.VMEM),
            pl.BlockSpec(memory_space=pltpu.SMEM),
            pl.BlockSpec(memory_space=pltpu.VMEM),
            pl.BlockSpec(memory_space=pltpu.VMEM),
            pl.BlockSpec(memory_space=pltpu.VMEM),
        ],
        out_shape=jax.ShapeDtypeStruct((n, 2 * hid), jnp.float32),
    )(p, h, beta, W1a, W1b, b1.reshape(1, -1))


def _sc_aggregate(h, meta, didx_all, zeros, nch):
    n, d = h.shape
    epw = meta.shape[0] * CH2 // NW

    mesh = plsc.VectorSubcoreMesh(core_axis_name="c", subcore_axis_name="s")

    @functools.partial(
        pl.kernel,
        out_type=jax.ShapeDtypeStruct((NC, n, d), jnp.float32),
        mesh=mesh,
        compiler_params=pltpu.CompilerParams(needs_layout_passes=False),
        scratch_types=[
            pltpu.VMEM_SHARED((n, d), jnp.float32),
        ]
        + [pltpu.VMEM((2, CH2), jnp.int32) for _ in range(RING)]
        + [pltpu.VMEM((CH2,), jnp.int32) for _ in range(RING)]
        + [pltpu.VMEM((CH2, d), jnp.float32) for _ in range(RING)]
        + [pltpu.SemaphoreType.DMA for _ in range(2 * RING)],
    )
    def k(h_hbm, meta_hbm, didx_hbm, z_hbm, part_hbm, acc_sh, *bufs):
        mv = bufs[0:RING]
        dv = bufs[RING:2 * RING]
        rows = bufs[2 * RING:3 * RING]
        gs = bufs[3 * RING:4 * RING]
        ss = bufs[4 * RING:5 * RING]
        c_ax = lax.axis_index("c")
        s_ax = lax.axis_index("s")
        wid = c_ax * NS + s_ax
        slab = n // NS
        row0 = s_ax * slab
        pltpu.sync_copy(z_hbm.at[pl.ds(row0, slab)],
                        acc_sh.at[pl.ds(row0, slab)])
        plsc.subcore_barrier()

        one = jnp.ones((LN,), jnp.int32)

        def issue(cc, b):
            base = wid * epw + cc * CH2
            pltpu.sync_copy(meta_hbm.at[wid * nch + cc], mv[b])
            pltpu.sync_copy(didx_hbm.at[pl.ds(base, CH2)], dv[b])
            pltpu.async_copy(h_hbm.at[mv[b].at[0]], rows[b], gs[b])

        issue(0, 0)
        issue(1, 1)

        def outer(g, carry):
            for b in range(RING):
                cc = g * RING + b
                pltpu.make_async_copy(h_hbm.at[mv[b].at[0]], rows[b],
                                      gs[b]).wait()

                def scale(i, _):
                    splat = jnp.zeros((LN,), jnp.int32) + i
                    w = plsc.bitcast(plsc.load_gather(mv[b], [one, splat]),
                                     jnp.float32)
                    for r in range(d // LN):
                        rows[b][i, pl.ds(r * LN, LN)] = (
                            rows[b][i, pl.ds(r * LN, LN)] * w
                        )
                    return _

                lax.fori_loop(0, CH2, scale, 0, unroll=2)
                pltpu.async_copy(rows[b], acc_sh.at[dv[b]], ss[b], add=True)

                b2 = (b + 2) % RING

                @pl.when(cc + 2 < nch)
                def _():
                    @pl.when(cc + 2 >= RING)
                    def _():
                        pltpu.make_async_copy(rows[b2], acc_sh.at[dv[b2]],
                                              ss[b2]).wait()
                    issue(cc + 2, b2)
            return carry

        lax.fori_loop(0, nch // RING, outer, 0)
        for b in range(RING):
            pltpu.make_async_copy(rows[b], acc_sh.at[dv[b]], ss[b]).wait()
        plsc.subcore_barrier()
        pltpu.sync_copy(acc_sh.at[pl.ds(row0, slab)],
                        part_hbm.at[c_ax, pl.ds(row0, slab)])

    return k(h, meta, didx_all, zeros)


def _sc_edge_logits(AB, meta, w2, b2, nch):
    n, two_hid = AB.shape
    hid = two_hid // 2
    epw = meta.shape[1] // NW
    e = meta.shape[1]

    mesh = plsc.VectorSubcoreMesh(core_axis_name="c", subcore_axis_name="s")

    @functools.partial(
        pl.kernel,
        out_type=jax.ShapeDtypeStruct((e,), jnp.float32),
        mesh=mesh,
        compiler_params=pltpu.CompilerParams(needs_layout_passes=False),
        scratch_types=[
            pltpu.VMEM((hid,), jnp.float32),
            pltpu.VMEM((16,), jnp.float32),
        ]
        + [pltpu.VMEM((2, CH4), jnp.int32) for _ in range(RING)]
        + [pltpu.VMEM((CH4, two_hid), jnp.float32) for _ in range(RING)]
        + [pltpu.VMEM((CH4, two_hid), jnp.float32) for _ in range(RING)]
        + [pltpu.VMEM((CH4,), jnp.float32) for _ in range(RING)]
        + [pltpu.SemaphoreType.DMA for _ in range(4 * RING)],
    )
    def k(ab_hbm, meta_hbm, w2_hbm, b2_hbm, out_hbm, w2v, b2v, *bufs):
        mv = bufs[0:RING]
        arows = bufs[RING:2 * RING]
        brows = bufs[2 * RING:3 * RING]
        outv = bufs[3 * RING:4 * RING]
        sa = bufs[4 * RING:5 * RING]
        sb = bufs[5 * RING:6 * RING]
        os_ = bufs[6 * RING:7 * RING]
        c_ax = lax.axis_index("c")
        s_ax = lax.axis_index("s")
        wid = c_ax * NS + s_ax
        pltpu.sync_copy(w2_hbm, w2v)
        pltpu.sync_copy(b2_hbm, b2v)
        w2r = [w2v[pl.ds(r * LN, LN)] for r in range(hid // LN)]
        b2vec = b2v[pl.ds(0, LN)]
        lane = lax.iota(jnp.int32, LN)

        def issue(cc, b):
            base = wid * epw + cc * CH4
            pltpu.sync_copy(meta_hbm.at[:, pl.ds(base, CH4)], mv[b])
            pltpu.async_copy(ab_hbm.at[mv[b].at[0]], arows[b], sa[b])
            pltpu.async_copy(ab_hbm.at[mv[b].at[1]], brows[b], sb[b])

        issue(0, 0)
        issue(1, 1)

        def outer(g, carry):
            for b in range(RING):
                cc = g * RING + b
                base = wid * epw + cc * CH4
                pltpu.make_async_copy(ab_hbm.at[mv[b].at[0]], arows[b],
                                      sa[b]).wait()
                pltpu.make_async_copy(ab_hbm.at[mv[b].at[1]], brows[b],
                                      sb[b]).wait()

                @pl.when(cc >= RING)
                def _():
                    pltpu.make_async_copy(outv[b],
                                          out_hbm.at[pl.ds(base, CH4)],
                                          os_[b]).wait()

                def group(gg, _):
                    acc = b2vec
                    for j in range(LN):
                        i = gg * LN + j
                        t = None
                        for r in range(hid // LN):
                            v = jnp.maximum(
                                arows[b][i, pl.ds(r * LN, LN)]
                                + brows[b][i, pl.ds(hid + r * LN, LN)],
                                0.0,
                            ) * w2r[r]
                            t = v if t is None else t + v
                        acc = jnp.where(lane == j, acc + jnp.sum(t), acc)
                    outv[b][pl.ds(gg * LN, LN)] = acc
                    return _

                lax.fori_loop(0, CH4 // LN, group, 0)
                pltpu.async_copy(outv[b], out_hbm.at[pl.ds(base, CH4)], os_[b])

                b2_ = (b + 2) % RING

                @pl.when(cc + 2 < nch)
                def _():
                    issue(cc + 2, b2_)
            return carry

        lax.fori_loop(0, nch // RING, outer, 0)
        for b in range(RING):
            base = wid * epw
            pltpu.make_async_copy(outv[b], out_hbm.at[pl.ds(base, CH4)],
                                  os_[b]).wait()

    return k(AB, meta, w2, b2)


def kernel(batch, x, edge_index, beta, edge_attr, edge_weight,
           W_enc, b_enc, W1, b1, W2, b2):
    n, d = x.shape
    e = edge_index.shape[1]
    src = edge_index[0]
    dst = edge_index[1]

    np_ = ((n + NS * 8 - 1) // (NS * 8)) * (NS * 8)
    x = jnp.pad(x, ((0, np_ - n), (0, 0)))

    def _pad_edges(ch):
        nch = -(-e // (NW * ch))
        nch = ((nch + RING - 1) // RING) * RING
        ep = nch * ch * NW
        return nch, ep

    nch2, ep2 = _pad_edges(CH2)
    src2 = jnp.pad(src, (0, ep2 - e))
    dst2 = jnp.pad(dst, (0, ep2 - e))
    ew2 = jnp.pad(edge_weight, (0, ep2 - e))
    meta2 = jnp.stack(
        [src2.reshape(-1, CH2),
         lax.bitcast_convert_type(ew2, jnp.int32).reshape(-1, CH2)],
        axis=1,
    )

    nch4, ep4 = _pad_edges(CH4)
    meta4 = jnp.stack([jnp.pad(src, (0, ep4 - e)), jnp.pad(dst, (0, ep4 - e))])

    h = _tc_encode(x, W_enc, b_enc)
    zeros = jnp.zeros((np_, d), dtype=jnp.float32)
    partials = _sc_aggregate(h, meta2, dst2, zeros, nch2)
    AB = _tc_node_mlp(partials, h, beta, W1[:d], W1[d:], b1)
    b2pad = jnp.full((16,), b2[0], jnp.float32)
    logits = _sc_edge_logits(AB, meta4, W2[:, 0], b2pad, nch4)
    return logits[:e].reshape(e, 1)


# speedup vs baseline: 1.5726x; 1.5726x over previous
import functools

import jax
import jax.numpy as jnp
from jax import lax
from jax.experimental import pallas as pl
from jax.experimental.pallas import tpu as pltpu
from jax.experimental.pallas import tpu_sc as plsc

NC = 2
NS = 16
LN = 16
NW = NC * NS

CH2 = 96
CH4 = 80
RING = 3


def _tc_encode(x, W_enc, b_enc):
    def body(x_ref, w_ref, b_ref, o_ref):
        o_ref[...] = (
            jnp.dot(x_ref[...], w_ref[...], preferred_element_type=jnp.float32)
            + b_ref[...]
        )

    return pl.pallas_call(
        body,
        out_shape=jax.ShapeDtypeStruct(x.shape, jnp.float32),
    )(x, W_enc, b_enc.reshape(1, -1))


def _tc_node_mlp(p, h, beta, W1a, W1b, b1):
    n, d = h.shape
    hid = W1a.shape[1]

    def body(p_ref, h_ref, beta_ref, wa_ref, wb_ref, b1_ref, ab_ref):
        ne = jnp.maximum(p_ref[0] + p_ref[1] + beta_ref[0] * h_ref[...], 0.0)
        a = jnp.dot(ne, wa_ref[...], preferred_element_type=jnp.float32) + b1_ref[...]
        b = jnp.dot(ne, wb_ref[...], preferred_element_type=jnp.float32)
        ab_ref[...] = jnp.concatenate([a, b], axis=1)

    return pl.pallas_call(
        body,
        in_specs=[
            pl.BlockSpec(memory_space=pltpu.VMEM),
            pl.BlockSpec(memory_space=pltpu.VMEM),
            pl.BlockSpec(memory_space=pltpu.SMEM),
            pl.BlockSpec(memory_space=pltpu.VMEM),
            pl.BlockSpec(memory_space=pltpu.VMEM),
            pl.BlockSpec(memory_space=pltpu.VMEM),
        ],
        out_shape=jax.ShapeDtypeStruct((n, 2 * hid), jnp.float32),
    )(p, h, beta, W1a, W1b, b1.reshape(1, -1))


def _sc_aggregate(h, meta, didx_all, zeros, nch):
    n, d = h.shape
    epw = meta.shape[0] * CH2 // NW

    mesh = plsc.VectorSubcoreMesh(core_axis_name="c", subcore_axis_name="s")

    @functools.partial(
        pl.kernel,
        out_type=jax.ShapeDtypeStruct((NC, n, d), jnp.float32),
        mesh=mesh,
        compiler_params=pltpu.CompilerParams(needs_layout_passes=False),
        scratch_types=[
            pltpu.VMEM_SHARED((n, d), jnp.float32),
        ]
        + [pltpu.VMEM((2, CH2), jnp.int32) for _ in range(RING)]
        + [pltpu.VMEM((CH2,), jnp.int32) for _ in range(RING)]
        + [pltpu.VMEM((CH2, d), jnp.float32) for _ in range(RING)]
        + [pltpu.SemaphoreType.DMA for _ in range(2 * RING)],
    )
    def k(h_hbm, meta_hbm, didx_hbm, z_hbm, part_hbm, acc_sh, *bufs):
        mv = bufs[0:RING]
        dv = bufs[RING:2 * RING]
        rows = bufs[2 * RING:3 * RING]
        gs = bufs[3 * RING:4 * RING]
        ss = bufs[4 * RING:5 * RING]
        c_ax = lax.axis_index("c")
        s_ax = lax.axis_index("s")
        wid = c_ax * NS + s_ax
        slab = n // NS
        row0 = s_ax * slab
        pltpu.sync_copy(z_hbm.at[pl.ds(row0, slab)],
                        acc_sh.at[pl.ds(row0, slab)])
        plsc.subcore_barrier()

        one = jnp.ones((LN,), jnp.int32)

        def issue(cc, b):
            base = wid * epw + cc * CH2
            pltpu.sync_copy(meta_hbm.at[wid * nch + cc], mv[b])
            pltpu.sync_copy(didx_hbm.at[pl.ds(base, CH2)], dv[b])
            pltpu.async_copy(h_hbm.at[mv[b].at[0]], rows[b], gs[b])

        issue(0, 0)
        issue(1, 1)

        def outer(g, carry):
            for b in range(RING):
                cc = g * RING + b
                pltpu.make_async_copy(h_hbm.at[mv[b].at[0]], rows[b],
                                      gs[b]).wait()

                def scale(i, _):
                    splat = jnp.zeros((LN,), jnp.int32) + i
                    w = plsc.bitcast(plsc.load_gather(mv[b], [one, splat]),
                                     jnp.float32)
                    for r in range(d // LN):
                        rows[b][i, pl.ds(r * LN, LN)] = (
                            rows[b][i, pl.ds(r * LN, LN)] * w
                        )
                    return _

                lax.fori_loop(0, CH2, scale, 0, unroll=2)
                pltpu.async_copy(rows[b], acc_sh.at[dv[b]], ss[b], add=True)

                b2 = (b + 2) % RING

                @pl.when(cc + 2 < nch)
                def _():
                    @pl.when(cc + 2 >= RING)
                    def _():
                        pltpu.make_async_copy(rows[b2], acc_sh.at[dv[b2]],
                                              ss[b2]).wait()
                    issue(cc + 2, b2)
            return carry

        lax.fori_loop(0, nch // RING, outer, 0)
        for b in range(RING):
            pltpu.make_async_copy(rows[b], acc_sh.at[dv[b]], ss[b]).wait()
        plsc.subcore_barrier()
        pltpu.sync_copy(acc_sh.at[pl.ds(row0, slab)],
                        part_hbm.at[c_ax, pl.ds(row0, slab)])

    return k(h, meta, didx_all, zeros)


def _sc_edge_logits(AB, src_all, dst_all, w2, b2, nch):
    n, two_hid = AB.shape
    hid = two_hid // 2
    e = src_all.shape[0]
    epw = e // NW

    mesh = plsc.VectorSubcoreMesh(core_axis_name="c", subcore_axis_name="s")

    @functools.partial(
        pl.kernel,
        out_type=jax.ShapeDtypeStruct((e,), jnp.float32),
        mesh=mesh,
        compiler_params=pltpu.CompilerParams(needs_layout_passes=False),
        scratch_types=[
            pltpu.VMEM((hid,), jnp.float32),
            pltpu.VMEM((16,), jnp.float32),
            pltpu.VMEM((CH4,), jnp.int32),
            pltpu.VMEM((CH4,), jnp.int32),
            pltpu.VMEM((CH4, two_hid), jnp.float32),
            pltpu.VMEM((CH4, two_hid), jnp.float32),
            pltpu.VMEM((CH4,), jnp.float32),
            pltpu.SemaphoreType.DMA,
            pltpu.SemaphoreType.DMA,
        ],
    )
    def k(ab_hbm, src_hbm, dst_hbm, w2_hbm, b2_hbm, out_hbm,
          w2v, b2v, sidx, didx, arows, brows, outv, sem_a, sem_b):
        c_ax = lax.axis_index("c")
        s_ax = lax.axis_index("s")
        wid = c_ax * NS + s_ax
        pltpu.sync_copy(w2_hbm, w2v)
        pltpu.sync_copy(b2_hbm, b2v)
        w2r = [w2v[pl.ds(r * LN, LN)] for r in range(hid // LN)]
        b2vec = b2v[pl.ds(0, LN)]
        lane = lax.iota(jnp.int32, LN)

        def chunk(cc, carry):
            base = wid * epw + cc * CH4
            pltpu.sync_copy(src_hbm.at[pl.ds(base, CH4)], sidx)
            pltpu.sync_copy(dst_hbm.at[pl.ds(base, CH4)], didx)
            ca = pltpu.async_copy(ab_hbm.at[sidx], arows, sem_a)
            cb = pltpu.async_copy(ab_hbm.at[didx], brows, sem_b)
            ca.wait()
            cb.wait()

            def group(gg, _):
                acc = b2vec
                for j in range(LN):
                    i = gg * LN + j
                    t = None
                    for r in range(hid // LN):
                        v = jnp.maximum(
                            arows[i, pl.ds(r * LN, LN)]
                            + brows[i, pl.ds(hid + r * LN, LN)],
                            0.0,
                        ) * w2r[r]
                        t = v if t is None else t + v
                    acc = jnp.where(lane == j, acc + jnp.sum(t), acc)
                outv[pl.ds(gg * LN, LN)] = acc
                return _

            lax.fori_loop(0, CH4 // LN, group, 0)
            pltpu.sync_copy(outv, out_hbm.at[pl.ds(base, CH4)])
            return carry

        lax.fori_loop(0, nch, chunk, 0)

    return k(AB, src_all, dst_all, w2, b2)


def kernel(batch, x, edge_index, beta, edge_attr, edge_weight,
           W_enc, b_enc, W1, b1, W2, b2):
    n, d = x.shape
    e = edge_index.shape[1]
    src = edge_index[0]
    dst = edge_index[1]

    np_ = ((n + NS * 8 - 1) // (NS * 8)) * (NS * 8)
    x = jnp.pad(x, ((0, np_ - n), (0, 0)))

    def _pad_edges(ch):
        nch = -(-e // (NW * ch))
        nch = ((nch + RING - 1) // RING) * RING
        ep = nch * ch * NW
        return nch, ep

    nch2, ep2 = _pad_edges(CH2)
    src2 = jnp.pad(src, (0, ep2 - e))
    dst2 = jnp.pad(dst, (0, ep2 - e))
    ew2 = jnp.pad(edge_weight, (0, ep2 - e))
    meta2 = jnp.stack(
        [src2.reshape(-1, CH2),
         lax.bitcast_convert_type(ew2, jnp.int32).reshape(-1, CH2)],
        axis=1,
    )

    nch4, ep4 = _pad_edges(CH4)
    src4 = jnp.pad(src, (0, ep4 - e))
    dst4 = jnp.pad(dst, (0, ep4 - e))

    h = _tc_encode(x, W_enc, b_enc)
    zeros = jnp.zeros((np_, d), dtype=jnp.float32)
    partials = _sc_aggregate(h, meta2, dst2, zeros, nch2)
    AB = _tc_node_mlp(partials, h, beta, W1[:d], W1[d:], b1)
    b2pad = jnp.full((16,), b2[0], jnp.float32)
    logits = _sc_edge_logits(AB, src4, dst4, W2[:, 0], b2pad, nch4)
    return logits[:e].reshape(e, 1)
